# baseline (speedup 1.0000x reference)
import functools

import jax
import jax.numpy as jnp
from jax import lax
from jax.experimental import pallas as pl
from jax.experimental.pallas import tpu as pltpu
from jax.experimental.pallas import tpu_sc as plsc

_N = 10000
_E = 320000
_D = 128
_H = 128
_G = 8
_MID = 64

_HH = _H // 2
_ROWS = 512
_NPAD = 10240
_NBLK = _NPAD // _ROWS
_TILES = 16
_CORES = 2
_NW = _TILES * _CORES
_CHUNK = 128
_CPT = 80
_EPT = _CPT * _CHUNK
_EPAD = _NW * _EPT
_DEGW = 8
_RPT = _NPAD // _TILES
_GRP = 8


def _sc_mesh():
    return plsc.VectorSubcoreMesh(core_axis_name="c", subcore_axis_name="s")


def _sc_degree(dst_t, ones_hb, zeros_hb):
    @functools.partial(
        pl.kernel,
        out_type=jax.ShapeDtypeStruct((_CORES, _NPAD, _DEGW), jnp.float32),
        mesh=_sc_mesh(),
        compiler_params=pltpu.CompilerParams(use_tc_tiling_on_sc=False),
        scratch_types=[
            pltpu.VMEM((_CPT, _CHUNK), jnp.int32),
            pltpu.VMEM((_CHUNK, _DEGW), jnp.float32),
            pltpu.VMEM((_RPT, _DEGW), jnp.float32),
            pltpu.VMEM_SHARED((_NPAD, _DEGW), jnp.float32),
        ],
    )
    def deg_kernel(dst_hbm, ones_hbm, zeros_hbm, out_hbm, idx_v, ones_v, zero_v, acc_sh):
        cid = lax.axis_index("c")
        sid = lax.axis_index("s")
        wid = cid * _TILES + sid
        r0 = sid * _RPT
        pltpu.sync_copy(dst_hbm.at[wid], idx_v)
        pltpu.sync_copy(ones_hbm, ones_v)
        pltpu.sync_copy(zeros_hbm, zero_v)
        pltpu.sync_copy(zero_v, acc_sh.at[pl.ds(r0, _RPT)])
        plsc.subcore_barrier()

        def body(j, carry):
            pltpu.sync_copy(ones_v, acc_sh.at[idx_v.at[j]], add=True)
            return carry

        lax.fori_loop(0, _CPT, body, 0)
        plsc.subcore_barrier()
        pltpu.sync_copy(acc_sh.at[pl.ds(r0, _RPT)], out_hbm.at[cid, pl.ds(r0, _RPT)])

    return deg_kernel(dst_t, ones_hb, zeros_hb)


def _sc_scatter(xs2, src_t, dst_t, zeros_hb):
    @functools.partial(
        pl.kernel,
        out_type=jax.ShapeDtypeStruct((_CORES, _NPAD, _HH), jnp.float32),
        mesh=_sc_mesh(),
        compiler_params=pltpu.CompilerParams(use_tc_tiling_on_sc=False),
        scratch_types=[
            pltpu.VMEM((_CPT, _CHUNK), jnp.int32),
            pltpu.VMEM((_CPT, _CHUNK), jnp.int32),
            pltpu.VMEM((_GRP * _CHUNK, _HH), jnp.float32),
            pltpu.VMEM_SHARED((_NPAD, _HH), jnp.float32),
            pltpu.SemaphoreType.DMA,
        ],
    )
    def scat_kernel(xs_hbm, src_hbm, dst_hbm, z_hbm, out_hbm,
                    src_v, dst_v, rows_v, acc_sh, sem):
        cid = lax.axis_index("c")
        sid = lax.axis_index("s")
        wid = cid * _TILES + sid
        r0 = sid * _RPT
        pltpu.sync_copy(src_hbm.at[wid], src_v)
        pltpu.sync_copy(dst_hbm.at[wid], dst_v)
        pltpu.sync_copy(z_hbm, rows_v.at[pl.ds(0, _CHUNK)])
        for kk in range(_RPT // _CHUNK):
            pltpu.sync_copy(rows_v.at[pl.ds(0, _CHUNK)],
                            acc_sh.at[pl.ds(r0 + kk * _CHUNK, _CHUNK)])
        plsc.subcore_barrier()

        def run(table):
            def grp(g, carry):
                base = g * _GRP
                cps = []
                for b in range(_GRP):
                    cp = pltpu.make_async_copy(
                        table.at[src_v.at[base + b]],
                        rows_v.at[pl.ds(b * _CHUNK, _CHUNK)], sem)
                    cp.start()
                    cps.append(cp)
                for cp in cps:
                    cp.wait()
                for b in range(_GRP):
                    pltpu.sync_copy(rows_v.at[pl.ds(b * _CHUNK, _CHUNK)],
                                    acc_sh.at[dst_v.at[base + b]], add=True)
                return carry

            lax.fori_loop(0, _CPT // _GRP, grp, 0)

        @pl.when(cid == 0)
        def _():
            run(xs_hbm.at[1])

        @pl.when(cid == 1)
        def _():
            run(xs_hbm.at[0])

        plsc.subcore_barrier()
        pltpu.sync_copy(acc_sh.at[pl.ds(r0, _RPT)],
                        out_hbm.at[1 - cid, pl.ds(r0, _RPT)])

    return scat_kernel(xs2, src_t, dst_t, zeros_hb)


def _dinv_block(dp0, dp1, block_idx):
    deg = dp0[:, 0:1] + dp1[:, 0:1] + 1.0
    dinv = lax.rsqrt(deg)
    rows = block_idx * _ROWS + lax.broadcasted_iota(jnp.int32, (_ROWS, 1), 0)
    return jnp.where(rows < _N, dinv, 0.0)


def _split_cols(y, o_ref):
    o_ref[0, :, :] = y[:, :_HH]
    o_ref[1, :, :] = y[:, _HH:]


_SPLIT_SPEC = pl.BlockSpec((2, _ROWS, _HH), lambda i: (0, i, 0))
_SPLIT_SHAPE = jax.ShapeDtypeStruct((2, _NPAD, _HH), jnp.float32)


def _tc_encode(x_pad, W_enc, b_enc8, Wc0, dp):
    def body(x_ref, we_ref, be_ref, wc_ref, dp_ref, o_ref):
        i = pl.program_id(0)
        h = jnp.dot(x_ref[...], we_ref[...], preferred_element_type=jnp.float32)
        h = h + be_ref[0:1, :]
        y = jnp.dot(h, wc_ref[...], preferred_element_type=jnp.float32)
        _split_cols(y * _dinv_block(dp_ref[0], dp_ref[1], i), o_ref)

    return pl.pallas_call(
        body,
        grid=(_NBLK,),
        in_specs=[
            pl.BlockSpec((_ROWS, _D), lambda i: (i, 0)),
            pl.BlockSpec((_D, _H), lambda i: (0, 0)),
            pl.BlockSpec((8, _H), lambda i: (0, 0)),
            pl.BlockSpec((_H, _H), lambda i: (0, 0)),
            pl.BlockSpec((_CORES, _ROWS, _DEGW), lambda i: (0, i, 0)),
        ],
        out_specs=_SPLIT_SPEC,
        out_shape=_SPLIT_SHAPE,
    )(x_pad, W_enc, b_enc8, Wc0, dp)


def _tc_post(p, xs2, dp, bias8):
    def body(p_ref, xs_ref, dp_ref, b_ref, y_ref, st_ref):
        i = pl.program_id(0)
        agg = jnp.concatenate([p_ref[0] + xs_ref[0], p_ref[1] + xs_ref[1]], axis=1)
        dinv = _dinv_block(dp_ref[0], dp_ref[1], i)
        rows = i * _ROWS + lax.broadcasted_iota(jnp.int32, (_ROWS, 1), 0)
        y = jnp.where(rows < _N, agg * dinv + b_ref[0:1, :], 0.0)
        y_ref[...] = y
        ps = jnp.sum(jnp.reshape(y, (_ROWS // 8, 8, _H)), axis=0)
        pq = jnp.sum(jnp.reshape(y * y, (_ROWS // 8, 8, _H)), axis=0)

        @pl.when(i == 0)
        def _():
            st_ref[0, :, :] = ps
            st_ref[1, :, :] = pq

        @pl.when(i > 0)
        def _():
            st_ref[0, :, :] += ps
            st_ref[1, :, :] += pq

    return pl.pallas_call(
        body,
        grid=(_NBLK,),
        in_specs=[
            _SPLIT_SPEC,
            _SPLIT_SPEC,
            pl.BlockSpec((_CORES, _ROWS, _DEGW), lambda i: (0, i, 0)),
            pl.BlockSpec((8, _H), lambda i: (0, 0)),
        ],
        out_specs=[
            pl.BlockSpec((_ROWS, _H), lambda i: (i, 0)),
            pl.BlockSpec((2, 8, _H), lambda i: (0, 0, 0)),
        ],
        out_shape=[
            jax.ShapeDtypeStruct((_NPAD, _H), jnp.float32),
            jax.ShapeDtypeStruct((2, 8, _H), jnp.float32),
        ],
    )(p, xs2, dp, bias8)


def _bn_from_stats(st_ref):
    s = jnp.sum(st_ref[0], axis=0, keepdims=True)
    q = jnp.sum(st_ref[1], axis=0, keepdims=True)
    m = s / _N
    v = q / _N - m * m
    return m, lax.rsqrt(v + 1e-5)


def _tc_norm_mm(y, st, g8, be8, Wc, dp):
    def body(y_ref, st_ref, g_ref, be_ref, w_ref, dp_ref, o_ref):
        i = pl.program_id(0)
        m, rstd = _bn_from_stats(st_ref)
        h = jnp.maximum((y_ref[...] - m) * rstd * g_ref[0:1, :] + be_ref[0:1, :], 0.0)
        o = jnp.dot(h, w_ref[...], preferred_element_type=jnp.float32)
        _split_cols(o * _dinv_block(dp_ref[0], dp_ref[1], i), o_ref)

    return pl.pallas_call(
        body,
        grid=(_NBLK,),
        in_specs=[
            pl.BlockSpec((_ROWS, _H), lambda i: (i, 0)),
            pl.BlockSpec((2, 8, _H), lambda i: (0, 0, 0)),
            pl.BlockSpec((8, _H), lambda i: (0, 0)),
            pl.BlockSpec((8, _H), lambda i: (0, 0)),
            pl.BlockSpec((_H, _H), lambda i: (0, 0)),
            pl.BlockSpec((_CORES, _ROWS, _DEGW), lambda i: (0, i, 0)),
        ],
        out_specs=_SPLIT_SPEC,
        out_shape=_SPLIT_SHAPE,
    )(y, st, g8, be8, Wc, dp)


def _tc_norm_pool(y, st, g8, be8, batch_t):
    def body(y_ref, st_ref, g_ref, be_ref, b_ref, o_ref):
        i = pl.program_id(0)
        m, rstd = _bn_from_stats(st_ref)
        h = jnp.maximum((y_ref[...] - m) * rstd * g_ref[0:1, :] + be_ref[0:1, :], 0.0)
        bids = b_ref[0]
        gids = lax.broadcasted_iota(jnp.int32, (_G, _ROWS), 0)
        oh = jnp.equal(gids, bids).astype(jnp.float32)
        psum = jnp.dot(oh, h, preferred_element_type=jnp.float32)
        pcnt = jnp.sum(oh, axis=1, keepdims=True) + jnp.zeros((_G, _H), jnp.float32)

        @pl.when(i == 0)
        def _():
            o_ref[0, :, :] = psum
            o_ref[1, :, :] = pcnt

        @pl.when(i > 0)
        def _():
            o_ref[0, :, :] += psum
            o_ref[1, :, :] += pcnt

    return pl.pallas_call(
        body,
        grid=(_NBLK,),
        in_specs=[
            pl.BlockSpec((_ROWS, _H), lambda i: (i, 0)),
            pl.BlockSpec((2, 8, _H), lambda i: (0, 0, 0)),
            pl.BlockSpec((8, _H), lambda i: (0, 0)),
            pl.BlockSpec((8, _H), lambda i: (0, 0)),
            pl.BlockSpec((1, 1, _ROWS), lambda i: (i, 0, 0)),
        ],
        out_specs=pl.BlockSpec((2, _G, _H), lambda i: (0, 0, 0)),
        out_shape=jax.ShapeDtypeStruct((2, _G, _H), jnp.float32),
    )(y, st, g8, be8, batch_t)


def _tc_head(pool, W1p, b1p, W2p, b2p):
    def body(po_ref, w1_ref, b1_ref, w2_ref, b2_ref, o_ref):
        pooled = po_ref[0] / jnp.maximum(po_ref[1], 1.0)
        hid = jnp.maximum(
            jnp.dot(pooled, w1_ref[...], preferred_element_type=jnp.float32) + b1_ref[...], 0.0)
        logit = jnp.dot(hid, w2_ref[...], preferred_element_type=jnp.float32) + b2_ref[...]
        o_ref[...] = jax.nn.sigmoid(logit)

    return pl.pallas_call(
        body,
        out_shape=jax.ShapeDtypeStruct((_G, _H), jnp.float32),
    )(pool, W1p, b1p, W2p, b2p)


def kernel(x, edge_index, batch, W_enc, b_enc, Wc0, bc0, Wc1, bc1,
           g0, be0, g1, be1, Wl1, bl1, Wl2, bl2):
    x_pad = jnp.pad(x, ((0, _NPAD - _N), (0, 0)))
    pad_e = _EPAD - _E
    src_t = jnp.concatenate(
        [edge_index[0], jnp.full((pad_e,), _N, jnp.int32)]).reshape(_NW, _CPT, _CHUNK)
    dst_t = jnp.concatenate(
        [edge_index[1], jnp.full((pad_e,), _N, jnp.int32)]).reshape(_NW, _CPT, _CHUNK)
    batch_t = jnp.concatenate(
        [batch, jnp.full((_NPAD - _N,), _G, jnp.int32)]).reshape(_NBLK, 1, _ROWS)
    ones_deg = jnp.ones((_CHUNK, _DEGW), jnp.float32)
    zeros_deg = jnp.zeros((_RPT, _DEGW), jnp.float32)
    zeros_row = jnp.zeros((_CHUNK, _HH), jnp.float32)
    b_enc8 = jnp.broadcast_to(b_enc, (8, _H))
    bc0_8 = jnp.broadcast_to(bc0, (8, _H))
    bc1_8 = jnp.broadcast_to(bc1, (8, _H))
    g0_8 = jnp.broadcast_to(g0, (8, _H))
    be0_8 = jnp.broadcast_to(be0, (8, _H))
    g1_8 = jnp.broadcast_to(g1, (8, _H))
    be1_8 = jnp.broadcast_to(be1, (8, _H))
    W1p = jnp.pad(Wl1, ((0, 0), (0, _H - _MID)))
    b1p = jnp.pad(jnp.broadcast_to(bl1, (_G, _MID)), ((0, 0), (0, _H - _MID)))
    W2p = jnp.pad(Wl2, ((0, _H - _MID), (0, _H - 1)))
    b2p = jnp.broadcast_to(bl2, (_G, _H))

    dp = _sc_degree(dst_t, ones_deg, zeros_deg)
    xs0 = _tc_encode(x_pad, W_enc, b_enc8, Wc0, dp)
    p0 = _sc_scatter(xs0, src_t, dst_t, zeros_row)
    y0, st0 = _tc_post(p0, xs0, dp, bc0_8)
    xs1 = _tc_norm_mm(y0, st0, g0_8, be0_8, Wc1, dp)
    p1 = _sc_scatter(xs1, src_t, dst_t, zeros_row)
    y1, st1 = _tc_post(p1, xs1, dp, bc1_8)
    pool = _tc_norm_pool(y1, st1, g1_8, be1_8, batch_t)
    out = _tc_head(pool, W1p, b1p, W2p, b2p)
    return out[:, 0]


# speedup vs baseline: 1.0045x; 1.0045x over previous
import functools

import jax
import jax.numpy as jnp
from jax import lax
from jax.experimental import pallas as pl
from jax.experimental.pallas import tpu as pltpu
from jax.experimental.pallas import tpu_sc as plsc

_N = 10000
_E = 320000
_D = 128
_H = 128
_G = 8
_MID = 64

_HH = _H // 2
_ROWS = 512
_NPAD = 10240
_NBLK = _NPAD // _ROWS
_TILES = 16
_CORES = 2
_NW = _TILES * _CORES
_CHUNK = 128
_CPT = 80
_EPT = _CPT * _CHUNK
_EPAD = _NW * _EPT
_DEGW = 8
_RPT = _NPAD // _TILES
_GRP = 8


def _sc_mesh():
    return plsc.VectorSubcoreMesh(core_axis_name="c", subcore_axis_name="s")


def _sc_degree(dst_t, ones_hb, zeros_hb):
    @functools.partial(
        pl.kernel,
        out_type=jax.ShapeDtypeStruct((_CORES, _NPAD, _DEGW), jnp.float32),
        mesh=_sc_mesh(),
        compiler_params=pltpu.CompilerParams(use_tc_tiling_on_sc=False),
        scratch_types=[
            pltpu.VMEM((_CPT, _CHUNK), jnp.int32),
            pltpu.VMEM((_CHUNK, _DEGW), jnp.float32),
            pltpu.VMEM((_RPT, _DEGW), jnp.float32),
            pltpu.VMEM_SHARED((_NPAD, _DEGW), jnp.float32),
        ],
    )
    def deg_kernel(dst_hbm, ones_hbm, zeros_hbm, out_hbm, idx_v, ones_v, zero_v, acc_sh):
        cid = lax.axis_index("c")
        sid = lax.axis_index("s")
        wid = cid * _TILES + sid
        r0 = sid * _RPT
        pltpu.sync_copy(dst_hbm.at[wid], idx_v)
        pltpu.sync_copy(ones_hbm, ones_v)
        pltpu.sync_copy(zeros_hbm, zero_v)
        pltpu.sync_copy(zero_v, acc_sh.at[pl.ds(r0, _RPT)])
        plsc.subcore_barrier()

        def body(j, carry):
            pltpu.sync_copy(ones_v, acc_sh.at[idx_v.at[j]], add=True)
            return carry

        lax.fori_loop(0, _CPT, body, 0)
        plsc.subcore_barrier()
        pltpu.sync_copy(acc_sh.at[pl.ds(r0, _RPT)], out_hbm.at[cid, pl.ds(r0, _RPT)])

    return deg_kernel(dst_t, ones_hb, zeros_hb)


def _sc_scatter(xs2, src_t, dst_t, zeros_hb):
    @functools.partial(
        pl.kernel,
        out_type=jax.ShapeDtypeStruct((_CORES, _NPAD, _HH), jnp.float32),
        mesh=_sc_mesh(),
        compiler_params=pltpu.CompilerParams(use_tc_tiling_on_sc=False),
        scratch_types=[
            pltpu.VMEM((_CPT, _CHUNK), jnp.int32),
            pltpu.VMEM((_CPT, _CHUNK), jnp.int32),
            pltpu.VMEM((_GRP * _CHUNK, _HH), jnp.float32),
            pltpu.VMEM_SHARED((_NPAD, _HH), jnp.float32),
            pltpu.SemaphoreType.DMA,
        ],
    )
    def scat_kernel(xs_hbm, src_hbm, dst_hbm, z_hbm, out_hbm,
                    src_v, dst_v, rows_v, acc_sh, sem):
        cid = lax.axis_index("c")
        sid = lax.axis_index("s")
        wid = cid * _TILES + sid
        r0 = sid * _RPT
        pltpu.sync_copy(src_hbm.at[wid], src_v)
        pltpu.sync_copy(dst_hbm.at[wid], dst_v)
        pltpu.sync_copy(z_hbm, rows_v.at[pl.ds(0, _CHUNK)])
        for kk in range(_RPT // _CHUNK):
            pltpu.sync_copy(rows_v.at[pl.ds(0, _CHUNK)],
                            acc_sh.at[pl.ds(r0 + kk * _CHUNK, _CHUNK)])
        plsc.subcore_barrier()

        def run(table):
            def grp(g, carry):
                base = g * _GRP
                cps = []
                for b in range(_GRP):
                    cp = pltpu.make_async_copy(
                        table.at[src_v.at[base + b]],
                        rows_v.at[pl.ds(b * _CHUNK, _CHUNK)], sem)
                    cp.start()
                    cps.append(cp)
                for cp in cps:
                    cp.wait()
                for b in range(_GRP):
                    pltpu.sync_copy(rows_v.at[pl.ds(b * _CHUNK, _CHUNK)],
                                    acc_sh.at[dst_v.at[base + b]], add=True)
                return carry

            lax.fori_loop(0, _CPT // _GRP, grp, 0)

        @pl.when(cid == 0)
        def _():
            run(xs_hbm.at[0])

        @pl.when(cid == 1)
        def _():
            run(xs_hbm.at[1])

        plsc.subcore_barrier()
        pltpu.sync_copy(acc_sh.at[pl.ds(r0, _RPT)], out_hbm.at[cid, pl.ds(r0, _RPT)])

    return scat_kernel(xs2, src_t, dst_t, zeros_hb)


def _dinv_block(dp0, dp1, block_idx):
    deg = dp0[:, 0:1] + dp1[:, 0:1] + 1.0
    dinv = lax.rsqrt(deg)
    rows = block_idx * _ROWS + lax.broadcasted_iota(jnp.int32, (_ROWS, 1), 0)
    return jnp.where(rows < _N, dinv, 0.0)


def _split_cols(y, o_ref):
    o_ref[0, :, :] = y[:, :_HH]
    o_ref[1, :, :] = y[:, _HH:]


_SPLIT_SPEC = pl.BlockSpec((2, _ROWS, _HH), lambda i: (0, i, 0))
_SPLIT_SHAPE = jax.ShapeDtypeStruct((2, _NPAD, _HH), jnp.float32)


def _tc_encode(x_pad, W_enc, b_enc8, Wc0, dp):
    def body(x_ref, we_ref, be_ref, wc_ref, dp_ref, o_ref):
        i = pl.program_id(0)
        h = jnp.dot(x_ref[...], we_ref[...], preferred_element_type=jnp.float32)
        h = h + be_ref[0:1, :]
        y = jnp.dot(h, wc_ref[...], preferred_element_type=jnp.float32)
        _split_cols(y * _dinv_block(dp_ref[0], dp_ref[1], i), o_ref)

    return pl.pallas_call(
        body,
        grid=(_NBLK,),
        in_specs=[
            pl.BlockSpec((_ROWS, _D), lambda i: (i, 0)),
            pl.BlockSpec((_D, _H), lambda i: (0, 0)),
            pl.BlockSpec((8, _H), lambda i: (0, 0)),
            pl.BlockSpec((_H, _H), lambda i: (0, 0)),
            pl.BlockSpec((_CORES, _ROWS, _DEGW), lambda i: (0, i, 0)),
        ],
        out_specs=_SPLIT_SPEC,
        out_shape=_SPLIT_SHAPE,
    )(x_pad, W_enc, b_enc8, Wc0, dp)


def _tc_post(p, xs2, dp, bias8):
    def body(p_ref, xs_ref, dp_ref, b_ref, y_ref, st_ref):
        i = pl.program_id(0)
        agg = jnp.concatenate([p_ref[0] + xs_ref[0], p_ref[1] + xs_ref[1]], axis=1)
        dinv = _dinv_block(dp_ref[0], dp_ref[1], i)
        rows = i * _ROWS + lax.broadcasted_iota(jnp.int32, (_ROWS, 1), 0)
        y = jnp.where(rows < _N, agg * dinv + b_ref[0:1, :], 0.0)
        y_ref[...] = y
        ps = jnp.sum(jnp.reshape(y, (_ROWS // 8, 8, _H)), axis=0)
        pq = jnp.sum(jnp.reshape(y * y, (_ROWS // 8, 8, _H)), axis=0)

        @pl.when(i == 0)
        def _():
            st_ref[0, :, :] = ps
            st_ref[1, :, :] = pq

        @pl.when(i > 0)
        def _():
            st_ref[0, :, :] += ps
            st_ref[1, :, :] += pq

    return pl.pallas_call(
        body,
        grid=(_NBLK,),
        in_specs=[
            _SPLIT_SPEC,
            _SPLIT_SPEC,
            pl.BlockSpec((_CORES, _ROWS, _DEGW), lambda i: (0, i, 0)),
            pl.BlockSpec((8, _H), lambda i: (0, 0)),
        ],
        out_specs=[
            pl.BlockSpec((_ROWS, _H), lambda i: (i, 0)),
            pl.BlockSpec((2, 8, _H), lambda i: (0, 0, 0)),
        ],
        out_shape=[
            jax.ShapeDtypeStruct((_NPAD, _H), jnp.float32),
            jax.ShapeDtypeStruct((2, 8, _H), jnp.float32),
        ],
    )(p, xs2, dp, bias8)


def _bn_from_stats(st_ref):
    s = jnp.sum(st_ref[0], axis=0, keepdims=True)
    q = jnp.sum(st_ref[1], axis=0, keepdims=True)
    m = s / _N
    v = q / _N - m * m
    return m, lax.rsqrt(v + 1e-5)


def _tc_norm_mm(y, st, g8, be8, Wc, dp):
    def body(y_ref, st_ref, g_ref, be_ref, w_ref, dp_ref, o_ref):
        i = pl.program_id(0)
        m, rstd = _bn_from_stats(st_ref)
        h = jnp.maximum((y_ref[...] - m) * rstd * g_ref[0:1, :] + be_ref[0:1, :], 0.0)
        o = jnp.dot(h, w_ref[...], preferred_element_type=jnp.float32)
        _split_cols(o * _dinv_block(dp_ref[0], dp_ref[1], i), o_ref)

    return pl.pallas_call(
        body,
        grid=(_NBLK,),
        in_specs=[
            pl.BlockSpec((_ROWS, _H), lambda i: (i, 0)),
            pl.BlockSpec((2, 8, _H), lambda i: (0, 0, 0)),
            pl.BlockSpec((8, _H), lambda i: (0, 0)),
            pl.BlockSpec((8, _H), lambda i: (0, 0)),
            pl.BlockSpec((_H, _H), lambda i: (0, 0)),
            pl.BlockSpec((_CORES, _ROWS, _DEGW), lambda i: (0, i, 0)),
        ],
        out_specs=_SPLIT_SPEC,
        out_shape=_SPLIT_SHAPE,
    )(y, st, g8, be8, Wc, dp)


def _tc_norm_pool(y, st, g8, be8, batch_t):
    def body(y_ref, st_ref, g_ref, be_ref, b_ref, o_ref):
        i = pl.program_id(0)
        m, rstd = _bn_from_stats(st_ref)
        h = jnp.maximum((y_ref[...] - m) * rstd * g_ref[0:1, :] + be_ref[0:1, :], 0.0)
        bids = b_ref[0]
        gids = lax.broadcasted_iota(jnp.int32, (_G, _ROWS), 0)
        oh = jnp.equal(gids, bids).astype(jnp.float32)
        psum = jnp.dot(oh, h, preferred_element_type=jnp.float32)
        pcnt = jnp.sum(oh, axis=1, keepdims=True) + jnp.zeros((_G, _H), jnp.float32)

        @pl.when(i == 0)
        def _():
            o_ref[0, :, :] = psum
            o_ref[1, :, :] = pcnt

        @pl.when(i > 0)
        def _():
            o_ref[0, :, :] += psum
            o_ref[1, :, :] += pcnt

    return pl.pallas_call(
        body,
        grid=(_NBLK,),
        in_specs=[
            pl.BlockSpec((_ROWS, _H), lambda i: (i, 0)),
            pl.BlockSpec((2, 8, _H), lambda i: (0, 0, 0)),
            pl.BlockSpec((8, _H), lambda i: (0, 0)),
            pl.BlockSpec((8, _H), lambda i: (0, 0)),
            pl.BlockSpec((1, 1, _ROWS), lambda i: (i, 0, 0)),
        ],
        out_specs=pl.BlockSpec((2, _G, _H), lambda i: (0, 0, 0)),
        out_shape=jax.ShapeDtypeStruct((2, _G, _H), jnp.float32),
    )(y, st, g8, be8, batch_t)


def _tc_head(pool, W1p, b1p, W2p, b2p):
    def body(po_ref, w1_ref, b1_ref, w2_ref, b2_ref, o_ref):
        pooled = po_ref[0] / jnp.maximum(po_ref[1], 1.0)
        hid = jnp.maximum(
            jnp.dot(pooled, w1_ref[...], preferred_element_type=jnp.float32) + b1_ref[...], 0.0)
        logit = jnp.dot(hid, w2_ref[...], preferred_element_type=jnp.float32) + b2_ref[...]
        o_ref[...] = jax.nn.sigmoid(logit)

    return pl.pallas_call(
        body,
        out_shape=jax.ShapeDtypeStruct((_G, _H), jnp.float32),
    )(pool, W1p, b1p, W2p, b2p)


def kernel(x, edge_index, batch, W_enc, b_enc, Wc0, bc0, Wc1, bc1,
           g0, be0, g1, be1, Wl1, bl1, Wl2, bl2):
    x_pad = jnp.pad(x, ((0, _NPAD - _N), (0, 0)))
    pad_e = _EPAD - _E
    src_t = jnp.concatenate(
        [edge_index[0], jnp.full((pad_e,), _N, jnp.int32)]).reshape(_NW, _CPT, _CHUNK)
    dst_t = jnp.concatenate(
        [edge_index[1], jnp.full((pad_e,), _N, jnp.int32)]).reshape(_NW, _CPT, _CHUNK)
    batch_t = jnp.concatenate(
        [batch, jnp.full((_NPAD - _N,), _G, jnp.int32)]).reshape(_NBLK, 1, _ROWS)
    ones_deg = jnp.ones((_CHUNK, _DEGW), jnp.float32)
    zeros_deg = jnp.zeros((_RPT, _DEGW), jnp.float32)
    zeros_row = jnp.zeros((_CHUNK, _HH), jnp.float32)
    b_enc8 = jnp.broadcast_to(b_enc, (8, _H))
    bc0_8 = jnp.broadcast_to(bc0, (8, _H))
    bc1_8 = jnp.broadcast_to(bc1, (8, _H))
    g0_8 = jnp.broadcast_to(g0, (8, _H))
    be0_8 = jnp.broadcast_to(be0, (8, _H))
    g1_8 = jnp.broadcast_to(g1, (8, _H))
    be1_8 = jnp.broadcast_to(be1, (8, _H))
    W1p = jnp.pad(Wl1, ((0, 0), (0, _H - _MID)))
    b1p = jnp.pad(jnp.broadcast_to(bl1, (_G, _MID)), ((0, 0), (0, _H - _MID)))
    W2p = jnp.pad(Wl2, ((0, _H - _MID), (0, _H - 1)))
    b2p = jnp.broadcast_to(bl2, (_G, _H))

    dp = _sc_degree(dst_t, ones_deg, zeros_deg)
    xs0 = _tc_encode(x_pad, W_enc, b_enc8, Wc0, dp)
    p0 = _sc_scatter(xs0, src_t, dst_t, zeros_row)
    y0, st0 = _tc_post(p0, xs0, dp, bc0_8)
    xs1 = _tc_norm_mm(y0, st0, g0_8, be0_8, Wc1, dp)
    p1 = _sc_scatter(xs1, src_t, dst_t, zeros_row)
    y1, st1 = _tc_post(p1, xs1, dp, bc1_8)
    pool = _tc_norm_pool(y1, st1, g1_8, be1_8, batch_t)
    out = _tc_head(pool, W1p, b1p, W2p, b2p)
    return out[:, 0]


# speedup vs baseline: 1.0662x; 1.0614x over previous
import functools

import jax
import jax.numpy as jnp
from jax import lax
from jax.experimental import pallas as pl
from jax.experimental.pallas import tpu as pltpu
from jax.experimental.pallas import tpu_sc as plsc

_N = 10000
_E = 320000
_D = 128
_H = 128
_G = 8
_MID = 64

_HH = _H // 2
_ROWS = 512
_NPAD = 10240
_NBLK = _NPAD // _ROWS
_TILES = 16
_CORES = 2
_NW = _TILES * _CORES
_CHUNK = 128
_CPT = 80
_EPT = _CPT * _CHUNK
_EPAD = _NW * _EPT
_DEGW = 8
_RPT = _NPAD // _TILES
_GRP = 4
_NG = _CPT // _GRP


def _sc_mesh():
    return plsc.VectorSubcoreMesh(core_axis_name="c", subcore_axis_name="s")


def _sc_degree(dst_t, ones_hb, zeros_hb):
    @functools.partial(
        pl.kernel,
        out_type=jax.ShapeDtypeStruct((_CORES, _NPAD, _DEGW), jnp.float32),
        mesh=_sc_mesh(),
        compiler_params=pltpu.CompilerParams(use_tc_tiling_on_sc=False),
        scratch_types=[
            pltpu.VMEM((_CPT, _CHUNK), jnp.int32),
            pltpu.VMEM((_CHUNK, _DEGW), jnp.float32),
            pltpu.VMEM((_RPT, _DEGW), jnp.float32),
            pltpu.VMEM_SHARED((_NPAD, _DEGW), jnp.float32),
        ],
    )
    def deg_kernel(dst_hbm, ones_hbm, zeros_hbm, out_hbm, idx_v, ones_v, zero_v, acc_sh):
        cid = lax.axis_index("c")
        sid = lax.axis_index("s")
        wid = cid * _TILES + sid
        r0 = sid * _RPT
        pltpu.sync_copy(dst_hbm.at[wid], idx_v)
        pltpu.sync_copy(ones_hbm, ones_v)
        pltpu.sync_copy(zeros_hbm, zero_v)
        pltpu.sync_copy(zero_v, acc_sh.at[pl.ds(r0, _RPT)])
        plsc.subcore_barrier()

        def body(j, carry):
            pltpu.sync_copy(ones_v, acc_sh.at[idx_v.at[j]], add=True)
            return carry

        lax.fori_loop(0, _CPT, body, 0)
        plsc.subcore_barrier()
        pltpu.sync_copy(acc_sh.at[pl.ds(r0, _RPT)], out_hbm.at[cid, pl.ds(r0, _RPT)])

    return deg_kernel(dst_t, ones_hb, zeros_hb)


def _sc_scatter(xs2, src_t, dst_t):
    @functools.partial(
        pl.kernel,
        out_type=jax.ShapeDtypeStruct((_CORES, _NPAD, _HH), jnp.float32),
        mesh=_sc_mesh(),
        compiler_params=pltpu.CompilerParams(use_tc_tiling_on_sc=False),
        scratch_types=[
            pltpu.VMEM((_CPT, _CHUNK), jnp.int32),
            pltpu.VMEM((_CPT, _CHUNK), jnp.int32),
            pltpu.VMEM((2 * _GRP * _CHUNK, _HH), jnp.float32),
            pltpu.VMEM_SHARED((_NPAD, _HH), jnp.float32),
            pltpu.SemaphoreType.DMA,
            pltpu.SemaphoreType.DMA,
        ],
    )
    def scat_kernel(xs_hbm, src_hbm, dst_hbm, out_hbm,
                    src_v, dst_v, rows_v, acc_sh, sem_a, sem_b):
        cid = lax.axis_index("c")
        sid = lax.axis_index("s")
        wid = cid * _TILES + sid
        r0 = sid * _RPT
        pltpu.sync_copy(src_hbm.at[wid], src_v)
        pltpu.sync_copy(dst_hbm.at[wid], dst_v)
        pltpu.sync_copy(xs_hbm.at[cid, pl.ds(r0, _RPT)], acc_sh.at[pl.ds(r0, _RPT)])
        plsc.subcore_barrier()

        def run(table):
            def fire(g, buf, sem):
                base = g * _GRP
                for b in range(_GRP):
                    pltpu.make_async_copy(
                        table.at[src_v.at[base + b]],
                        rows_v.at[pl.ds((buf * _GRP + b) * _CHUNK, _CHUNK)],
                        sem).start()

            fire(0, 0, sem_a)
            fire(1, 1, sem_b)

            def pair(gg, carry):
                ga = 2 * gg
                for b in range(_GRP):
                    pltpu.make_async_copy(
                        table.at[src_v.at[ga * _GRP + b]],
                        rows_v.at[pl.ds(b * _CHUNK, _CHUNK)], sem_a).wait()

                @pl.when(ga + 2 < _NG)
                def _():
                    fire(ga + 2, 0, sem_a)

                for b in range(_GRP):
                    pltpu.sync_copy(
                        rows_v.at[pl.ds(b * _CHUNK, _CHUNK)],
                        acc_sh.at[dst_v.at[ga * _GRP + b]], add=True)

                for b in range(_GRP):
                    pltpu.make_async_copy(
                        table.at[src_v.at[(ga + 1) * _GRP + b]],
                        rows_v.at[pl.ds((_GRP + b) * _CHUNK, _CHUNK)], sem_b).wait()

                @pl.when(ga + 3 < _NG)
                def _():
                    fire(ga + 3, 1, sem_b)

                for b in range(_GRP):
                    pltpu.sync_copy(
                        rows_v.at[pl.ds((_GRP + b) * _CHUNK, _CHUNK)],
                        acc_sh.at[dst_v.at[(ga + 1) * _GRP + b]], add=True)
                return carry

            lax.fori_loop(0, _NG // 2, pair, 0)

        @pl.when(cid == 0)
        def _():
            run(xs_hbm.at[0])

        @pl.when(cid == 1)
        def _():
            run(xs_hbm.at[1])

        plsc.subcore_barrier()
        pltpu.sync_copy(acc_sh.at[pl.ds(r0, _RPT)], out_hbm.at[cid, pl.ds(r0, _RPT)])

    return scat_kernel(xs2, src_t, dst_t)


def _dinv_block(dp0, dp1, block_idx):
    deg = dp0[:, 0:1] + dp1[:, 0:1] + 1.0
    dinv = lax.rsqrt(deg)
    rows = block_idx * _ROWS + lax.broadcasted_iota(jnp.int32, (_ROWS, 1), 0)
    return jnp.where(rows < _N, dinv, 0.0)


def _split_cols(y, o_ref):
    o_ref[0, :, :] = y[:, :_HH]
    o_ref[1, :, :] = y[:, _HH:]


_SPLIT_SPEC = pl.BlockSpec((2, _ROWS, _HH), lambda i: (0, i, 0))
_SPLIT_SHAPE = jax.ShapeDtypeStruct((2, _NPAD, _HH), jnp.float32)


def _tc_encode(x_pad, W_enc, b_enc8, Wc0, dp):
    def body(x_ref, we_ref, be_ref, wc_ref, dp_ref, o_ref):
        i = pl.program_id(0)
        h = jnp.dot(x_ref[...], we_ref[...], preferred_element_type=jnp.float32)
        h = h + be_ref[0:1, :]
        y = jnp.dot(h, wc_ref[...], preferred_element_type=jnp.float32)
        _split_cols(y * _dinv_block(dp_ref[0], dp_ref[1], i), o_ref)

    return pl.pallas_call(
        body,
        grid=(_NBLK,),
        in_specs=[
            pl.BlockSpec((_ROWS, _D), lambda i: (i, 0)),
            pl.BlockSpec((_D, _H), lambda i: (0, 0)),
            pl.BlockSpec((8, _H), lambda i: (0, 0)),
            pl.BlockSpec((_H, _H), lambda i: (0, 0)),
            pl.BlockSpec((_CORES, _ROWS, _DEGW), lambda i: (0, i, 0)),
        ],
        out_specs=_SPLIT_SPEC,
        out_shape=_SPLIT_SHAPE,
    )(x_pad, W_enc, b_enc8, Wc0, dp)


def _tc_post(p, dp, bias8):
    def body(p_ref, dp_ref, b_ref, y_ref, st_ref):
        i = pl.program_id(0)
        agg = jnp.concatenate([p_ref[0], p_ref[1]], axis=1)
        dinv = _dinv_block(dp_ref[0], dp_ref[1], i)
        rows = i * _ROWS + lax.broadcasted_iota(jnp.int32, (_ROWS, 1), 0)
        y = jnp.where(rows < _N, agg * dinv + b_ref[0:1, :], 0.0)
        y_ref[...] = y
        ps = jnp.sum(jnp.reshape(y, (_ROWS // 8, 8, _H)), axis=0)
        pq = jnp.sum(jnp.reshape(y * y, (_ROWS // 8, 8, _H)), axis=0)

        @pl.when(i == 0)
        def _():
            st_ref[0, :, :] = ps
            st_ref[1, :, :] = pq

        @pl.when(i > 0)
        def _():
            st_ref[0, :, :] += ps
            st_ref[1, :, :] += pq

    return pl.pallas_call(
        body,
        grid=(_NBLK,),
        in_specs=[
            _SPLIT_SPEC,
            pl.BlockSpec((_CORES, _ROWS, _DEGW), lambda i: (0, i, 0)),
            pl.BlockSpec((8, _H), lambda i: (0, 0)),
        ],
        out_specs=[
            pl.BlockSpec((_ROWS, _H), lambda i: (i, 0)),
            pl.BlockSpec((2, 8, _H), lambda i: (0, 0, 0)),
        ],
        out_shape=[
            jax.ShapeDtypeStruct((_NPAD, _H), jnp.float32),
            jax.ShapeDtypeStruct((2, 8, _H), jnp.float32),
        ],
    )(p, dp, bias8)


def _bn_from_stats(st_ref):
    s = jnp.sum(st_ref[0], axis=0, keepdims=True)
    q = jnp.sum(st_ref[1], axis=0, keepdims=True)
    m = s / _N
    v = q / _N - m * m
    return m, lax.rsqrt(v + 1e-5)


def _tc_norm_mm(y, st, g8, be8, Wc, dp):
    def body(y_ref, st_ref, g_ref, be_ref, w_ref, dp_ref, o_ref):
        i = pl.program_id(0)
        m, rstd = _bn_from_stats(st_ref)
        h = jnp.maximum((y_ref[...] - m) * rstd * g_ref[0:1, :] + be_ref[0:1, :], 0.0)
        o = jnp.dot(h, w_ref[...], preferred_element_type=jnp.float32)
        _split_cols(o * _dinv_block(dp_ref[0], dp_ref[1], i), o_ref)

    return pl.pallas_call(
        body,
        grid=(_NBLK,),
        in_specs=[
            pl.BlockSpec((_ROWS, _H), lambda i: (i, 0)),
            pl.BlockSpec((2, 8, _H), lambda i: (0, 0, 0)),
            pl.BlockSpec((8, _H), lambda i: (0, 0)),
            pl.BlockSpec((8, _H), lambda i: (0, 0)),
            pl.BlockSpec((_H, _H), lambda i: (0, 0)),
            pl.BlockSpec((_CORES, _ROWS, _DEGW), lambda i: (0, i, 0)),
        ],
        out_specs=_SPLIT_SPEC,
        out_shape=_SPLIT_SHAPE,
    )(y, st, g8, be8, Wc, dp)


def _tc_norm_pool(y, st, g8, be8, batch_t):
    def body(y_ref, st_ref, g_ref, be_ref, b_ref, o_ref):
        i = pl.program_id(0)
        m, rstd = _bn_from_stats(st_ref)
        h = jnp.maximum((y_ref[...] - m) * rstd * g_ref[0:1, :] + be_ref[0:1, :], 0.0)
        bids = b_ref[0]
        gids = lax.broadcasted_iota(jnp.int32, (_G, _ROWS), 0)
        oh = jnp.equal(gids, bids).astype(jnp.float32)
        psum = jnp.dot(oh, h, preferred_element_type=jnp.float32)
        pcnt = jnp.sum(oh, axis=1, keepdims=True) + jnp.zeros((_G, _H), jnp.float32)

        @pl.when(i == 0)
        def _():
            o_ref[0, :, :] = psum
            o_ref[1, :, :] = pcnt

        @pl.when(i > 0)
        def _():
            o_ref[0, :, :] += psum
            o_ref[1, :, :] += pcnt

    return pl.pallas_call(
        body,
        grid=(_NBLK,),
        in_specs=[
            pl.BlockSpec((_ROWS, _H), lambda i: (i, 0)),
            pl.BlockSpec((2, 8, _H), lambda i: (0, 0, 0)),
            pl.BlockSpec((8, _H), lambda i: (0, 0)),
            pl.BlockSpec((8, _H), lambda i: (0, 0)),
            pl.BlockSpec((1, 1, _ROWS), lambda i: (i, 0, 0)),
        ],
        out_specs=pl.BlockSpec((2, _G, _H), lambda i: (0, 0, 0)),
        out_shape=jax.ShapeDtypeStruct((2, _G, _H), jnp.float32),
    )(y, st, g8, be8, batch_t)


def _tc_head(pool, W1p, b1p, W2p, b2p):
    def body(po_ref, w1_ref, b1_ref, w2_ref, b2_ref, o_ref):
        pooled = po_ref[0] / jnp.maximum(po_ref[1], 1.0)
        hid = jnp.maximum(
            jnp.dot(pooled, w1_ref[...], preferred_element_type=jnp.float32) + b1_ref[...], 0.0)
        logit = jnp.dot(hid, w2_ref[...], preferred_element_type=jnp.float32) + b2_ref[...]
        o_ref[...] = jax.nn.sigmoid(logit)

    return pl.pallas_call(
        body,
        out_shape=jax.ShapeDtypeStruct((_G, _H), jnp.float32),
    )(pool, W1p, b1p, W2p, b2p)


def kernel(x, edge_index, batch, W_enc, b_enc, Wc0, bc0, Wc1, bc1,
           g0, be0, g1, be1, Wl1, bl1, Wl2, bl2):
    x_pad = jnp.pad(x, ((0, _NPAD - _N), (0, 0)))
    pad_e = _EPAD - _E
    src_t = jnp.concatenate(
        [edge_index[0], jnp.full((pad_e,), _N, jnp.int32)]).reshape(_NW, _CPT, _CHUNK)
    dst_t = jnp.concatenate(
        [edge_index[1], jnp.full((pad_e,), _N, jnp.int32)]).reshape(_NW, _CPT, _CHUNK)
    batch_t = jnp.concatenate(
        [batch, jnp.full((_NPAD - _N,), _G, jnp.int32)]).reshape(_NBLK, 1, _ROWS)
    ones_deg = jnp.ones((_CHUNK, _DEGW), jnp.float32)
    zeros_deg = jnp.zeros((_RPT, _DEGW), jnp.float32)
    b_enc8 = jnp.broadcast_to(b_enc, (8, _H))
    bc0_8 = jnp.broadcast_to(bc0, (8, _H))
    bc1_8 = jnp.broadcast_to(bc1, (8, _H))
    g0_8 = jnp.broadcast_to(g0, (8, _H))
    be0_8 = jnp.broadcast_to(be0, (8, _H))
    g1_8 = jnp.broadcast_to(g1, (8, _H))
    be1_8 = jnp.broadcast_to(be1, (8, _H))
    W1p = jnp.pad(Wl1, ((0, 0), (0, _H - _MID)))
    b1p = jnp.pad(jnp.broadcast_to(bl1, (_G, _MID)), ((0, 0), (0, _H - _MID)))
    W2p = jnp.pad(Wl2, ((0, _H - _MID), (0, _H - 1)))
    b2p = jnp.broadcast_to(bl2, (_G, _H))

    dp = _sc_degree(dst_t, ones_deg, zeros_deg)
    xs0 = _tc_encode(x_pad, W_enc, b_enc8, Wc0, dp)
    p0 = _sc_scatter(xs0, src_t, dst_t)
    y0, st0 = _tc_post(p0, dp, bc0_8)
    xs1 = _tc_norm_mm(y0, st0, g0_8, be0_8, Wc1, dp)
    p1 = _sc_scatter(xs1, src_t, dst_t)
    y1, st1 = _tc_post(p1, dp, bc1_8)
    pool = _tc_norm_pool(y1, st1, g1_8, be1_8, batch_t)
    out = _tc_head(pool, W1p, b1p, W2p, b2p)
    return out[:, 0]


# speedup vs baseline: 1.1269x; 1.0570x over previous
import functools

import jax
import jax.numpy as jnp
from jax import lax
from jax.experimental import pallas as pl
from jax.experimental.pallas import tpu as pltpu
from jax.experimental.pallas import tpu_sc as plsc

_N = 10000
_E = 320000
_D = 128
_H = 128
_G = 8
_MID = 64

_HH = _H // 2
_ROWS = 512
_NPAD = 10240
_NBLK = _NPAD // _ROWS
_TILES = 16
_CORES = 2
_NW = _TILES * _CORES
_CHUNK = 128
_CPT = 80
_EPT = _CPT * _CHUNK
_EPAD = _NW * _EPT
_DEGW = 8
_RPT = _NPAD // _TILES
_GRP = 4
_NG = _CPT // _GRP


def _sc_mesh():
    return plsc.VectorSubcoreMesh(core_axis_name="c", subcore_axis_name="s")


def _sc_degree(dst_t, ones_hb, zeros_hb):
    @functools.partial(
        pl.kernel,
        out_type=jax.ShapeDtypeStruct((_CORES, _NPAD, _DEGW), jnp.float32),
        mesh=_sc_mesh(),
        compiler_params=pltpu.CompilerParams(use_tc_tiling_on_sc=False),
        scratch_types=[
            pltpu.VMEM((_CPT, _CHUNK), jnp.int32),
            pltpu.VMEM((_CHUNK, _DEGW), jnp.float32),
            pltpu.VMEM((_RPT, _DEGW), jnp.float32),
            pltpu.VMEM_SHARED((_NPAD, _DEGW), jnp.float32),
        ],
    )
    def deg_kernel(dst_hbm, ones_hbm, zeros_hbm, out_hbm, idx_v, ones_v, zero_v, acc_sh):
        cid = lax.axis_index("c")
        sid = lax.axis_index("s")
        wid = cid * _TILES + sid
        r0 = sid * _RPT
        pltpu.sync_copy(dst_hbm.at[wid], idx_v)
        pltpu.sync_copy(ones_hbm, ones_v)
        pltpu.sync_copy(zeros_hbm, zero_v)
        pltpu.sync_copy(zero_v, acc_sh.at[pl.ds(r0, _RPT)])
        plsc.subcore_barrier()

        def body(j, carry):
            pltpu.sync_copy(ones_v, acc_sh.at[idx_v.at[j]], add=True)
            return carry

        lax.fori_loop(0, _CPT, body, 0)
        plsc.subcore_barrier()
        pltpu.sync_copy(acc_sh.at[pl.ds(r0, _RPT)], out_hbm.at[cid, pl.ds(r0, _RPT)])

    return deg_kernel(dst_t, ones_hb, zeros_hb)


def _sc_scatter(xs2, src_t, dst_t):
    @functools.partial(
        pl.kernel,
        out_type=jax.ShapeDtypeStruct((_CORES, _NPAD, _HH), jnp.float32),
        mesh=_sc_mesh(),
        compiler_params=pltpu.CompilerParams(use_tc_tiling_on_sc=False),
        scratch_types=[
            pltpu.VMEM((_CPT, _CHUNK), jnp.int32),
            pltpu.VMEM((_CPT, _CHUNK), jnp.int32),
            pltpu.VMEM((2 * _GRP * _CHUNK, _HH), jnp.float32),
            pltpu.VMEM_SHARED((_NPAD, _HH), jnp.float32),
            pltpu.SemaphoreType.DMA,
            pltpu.SemaphoreType.DMA,
            pltpu.SemaphoreType.DMA,
        ],
    )
    def scat_kernel(xs_hbm, src_hbm, dst_hbm, out_hbm,
                    src_v, dst_v, rows_v, acc_sh, sem_a, sem_b, sem_c):
        cid = lax.axis_index("c")
        sid = lax.axis_index("s")
        wid = cid * _TILES + sid
        r0 = sid * _RPT
        src_cp = pltpu.make_async_copy(src_hbm.at[wid], src_v, sem_a)
        src_cp.start()
        dst_cp = pltpu.make_async_copy(dst_hbm.at[wid], dst_v, sem_c)
        dst_cp.start()
        pre_cp = pltpu.make_async_copy(
            xs_hbm.at[cid, pl.ds(r0, _RPT)], acc_sh.at[pl.ds(r0, _RPT)], sem_c)
        pre_cp.start()

        table = xs_hbm.at[cid]

        def fire(g, buf, sem):
            base = g * _GRP
            for b in range(_GRP):
                pltpu.make_async_copy(
                    table.at[src_v.at[base + b]],
                    rows_v.at[pl.ds((buf * _GRP + b) * _CHUNK, _CHUNK)],
                    sem).start()

        src_cp.wait()
        fire(0, 0, sem_a)
        fire(1, 1, sem_b)
        dst_cp.wait()
        pre_cp.wait()
        plsc.subcore_barrier()

        def pair(gg, carry):
            ga = 2 * gg
            for b in range(_GRP):
                pltpu.make_async_copy(
                    table.at[src_v.at[ga * _GRP + b]],
                    rows_v.at[pl.ds(b * _CHUNK, _CHUNK)], sem_a).wait()

            @pl.when(ga + 2 < _NG)
            def _():
                fire(ga + 2, 0, sem_a)

            for b in range(_GRP):
                pltpu.sync_copy(
                    rows_v.at[pl.ds(b * _CHUNK, _CHUNK)],
                    acc_sh.at[dst_v.at[ga * _GRP + b]], add=True)

            for b in range(_GRP):
                pltpu.make_async_copy(
                    table.at[src_v.at[(ga + 1) * _GRP + b]],
                    rows_v.at[pl.ds((_GRP + b) * _CHUNK, _CHUNK)], sem_b).wait()

            @pl.when(ga + 3 < _NG)
            def _():
                fire(ga + 3, 1, sem_b)

            for b in range(_GRP):
                pltpu.sync_copy(
                    rows_v.at[pl.ds((_GRP + b) * _CHUNK, _CHUNK)],
                    acc_sh.at[dst_v.at[(ga + 1) * _GRP + b]], add=True)
            return carry

        lax.fori_loop(0, _NG // 2, pair, 0)

        plsc.subcore_barrier()
        pltpu.sync_copy(acc_sh.at[pl.ds(r0, _RPT)], out_hbm.at[cid, pl.ds(r0, _RPT)])

    return scat_kernel(xs2, src_t, dst_t)


def _dinv_block(dp0, dp1, block_idx):
    deg = dp0[:, 0:1] + dp1[:, 0:1] + 1.0
    dinv = lax.rsqrt(deg)
    rows = block_idx * _ROWS + lax.broadcasted_iota(jnp.int32, (_ROWS, 1), 0)
    return jnp.where(rows < _N, dinv, 0.0)


def _split_cols(y, o_ref):
    o_ref[0, :, :] = y[:, :_HH]
    o_ref[1, :, :] = y[:, _HH:]


_SPLIT_SPEC = pl.BlockSpec((2, _ROWS, _HH), lambda i: (0, i, 0))
_SPLIT_SHAPE = jax.ShapeDtypeStruct((2, _NPAD, _HH), jnp.float32)


def _tc_encode_mm(x_pad, W_enc, b_enc8, Wc0):
    def body(x_ref, we_ref, be_ref, wc_ref, o_ref):
        h = jnp.dot(x_ref[...], we_ref[...], preferred_element_type=jnp.float32)
        h = h + be_ref[0:1, :]
        o_ref[...] = jnp.dot(h, wc_ref[...], preferred_element_type=jnp.float32)

    return pl.pallas_call(
        body,
        grid=(_NBLK,),
        in_specs=[
            pl.BlockSpec((_ROWS, _D), lambda i: (i, 0)),
            pl.BlockSpec((_D, _H), lambda i: (0, 0)),
            pl.BlockSpec((8, _H), lambda i: (0, 0)),
            pl.BlockSpec((_H, _H), lambda i: (0, 0)),
        ],
        out_specs=pl.BlockSpec((_ROWS, _H), lambda i: (i, 0)),
        out_shape=jax.ShapeDtypeStruct((_NPAD, _H), jnp.float32),
    )(x_pad, W_enc, b_enc8, Wc0)


def _tc_scale_split(xh, dp):
    def body(xh_ref, dp_ref, o_ref):
        i = pl.program_id(0)
        _split_cols(xh_ref[...] * _dinv_block(dp_ref[0], dp_ref[1], i), o_ref)

    return pl.pallas_call(
        body,
        grid=(_NBLK,),
        in_specs=[
            pl.BlockSpec((_ROWS, _H), lambda i: (i, 0)),
            pl.BlockSpec((_CORES, _ROWS, _DEGW), lambda i: (0, i, 0)),
        ],
        out_specs=_SPLIT_SPEC,
        out_shape=_SPLIT_SHAPE,
    )(xh, dp)


def _tc_post(p, dp, bias8):
    def body(p_ref, dp_ref, b_ref, y_ref, st_ref):
        i = pl.program_id(0)
        agg = jnp.concatenate([p_ref[0], p_ref[1]], axis=1)
        dinv = _dinv_block(dp_ref[0], dp_ref[1], i)
        rows = i * _ROWS + lax.broadcasted_iota(jnp.int32, (_ROWS, 1), 0)
        y = jnp.where(rows < _N, agg * dinv + b_ref[0:1, :], 0.0)
        y_ref[...] = y
        ps = jnp.sum(jnp.reshape(y, (_ROWS // 8, 8, _H)), axis=0)
        pq = jnp.sum(jnp.reshape(y * y, (_ROWS // 8, 8, _H)), axis=0)

        @pl.when(i == 0)
        def _():
            st_ref[0, :, :] = ps
            st_ref[1, :, :] = pq

        @pl.when(i > 0)
        def _():
            st_ref[0, :, :] += ps
            st_ref[1, :, :] += pq

    return pl.pallas_call(
        body,
        grid=(_NBLK,),
        in_specs=[
            _SPLIT_SPEC,
            pl.BlockSpec((_CORES, _ROWS, _DEGW), lambda i: (0, i, 0)),
            pl.BlockSpec((8, _H), lambda i: (0, 0)),
        ],
        out_specs=[
            pl.BlockSpec((_ROWS, _H), lambda i: (i, 0)),
            pl.BlockSpec((2, 8, _H), lambda i: (0, 0, 0)),
        ],
        out_shape=[
            jax.ShapeDtypeStruct((_NPAD, _H), jnp.float32),
            jax.ShapeDtypeStruct((2, 8, _H), jnp.float32),
        ],
    )(p, dp, bias8)


def _bn_from_stats(st_ref):
    s = jnp.sum(st_ref[0], axis=0, keepdims=True)
    q = jnp.sum(st_ref[1], axis=0, keepdims=True)
    m = s / _N
    v = q / _N - m * m
    return m, lax.rsqrt(v + 1e-5)


def _tc_norm_mm(y, st, g8, be8, Wc, dp):
    def body(y_ref, st_ref, g_ref, be_ref, w_ref, dp_ref, o_ref):
        i = pl.program_id(0)
        m, rstd = _bn_from_stats(st_ref)
        h = jnp.maximum((y_ref[...] - m) * rstd * g_ref[0:1, :] + be_ref[0:1, :], 0.0)
        o = jnp.dot(h, w_ref[...], preferred_element_type=jnp.float32)
        _split_cols(o * _dinv_block(dp_ref[0], dp_ref[1], i), o_ref)

    return pl.pallas_call(
        body,
        grid=(_NBLK,),
        in_specs=[
            pl.BlockSpec((_ROWS, _H), lambda i: (i, 0)),
            pl.BlockSpec((2, 8, _H), lambda i: (0, 0, 0)),
            pl.BlockSpec((8, _H), lambda i: (0, 0)),
            pl.BlockSpec((8, _H), lambda i: (0, 0)),
            pl.BlockSpec((_H, _H), lambda i: (0, 0)),
            pl.BlockSpec((_CORES, _ROWS, _DEGW), lambda i: (0, i, 0)),
        ],
        out_specs=_SPLIT_SPEC,
        out_shape=_SPLIT_SHAPE,
    )(y, st, g8, be8, Wc, dp)


def _tc_norm_pool(y, st, g8, be8, batch_t):
    def body(y_ref, st_ref, g_ref, be_ref, b_ref, o_ref):
        i = pl.program_id(0)
        m, rstd = _bn_from_stats(st_ref)
        h = jnp.maximum((y_ref[...] - m) * rstd * g_ref[0:1, :] + be_ref[0:1, :], 0.0)
        bids = b_ref[0]
        gids = lax.broadcasted_iota(jnp.int32, (_G, _ROWS), 0)
        oh = jnp.equal(gids, bids).astype(jnp.float32)
        psum = jnp.dot(oh, h, preferred_element_type=jnp.float32)
        pcnt = jnp.sum(oh, axis=1, keepdims=True) + jnp.zeros((_G, _H), jnp.float32)

        @pl.when(i == 0)
        def _():
            o_ref[0, :, :] = psum
            o_ref[1, :, :] = pcnt

        @pl.when(i > 0)
        def _():
            o_ref[0, :, :] += psum
            o_ref[1, :, :] += pcnt

    return pl.pallas_call(
        body,
        grid=(_NBLK,),
        in_specs=[
            pl.BlockSpec((_ROWS, _H), lambda i: (i, 0)),
            pl.BlockSpec((2, 8, _H), lambda i: (0, 0, 0)),
            pl.BlockSpec((8, _H), lambda i: (0, 0)),
            pl.BlockSpec((8, _H), lambda i: (0, 0)),
            pl.BlockSpec((1, 1, _ROWS), lambda i: (i, 0, 0)),
        ],
        out_specs=pl.BlockSpec((2, _G, _H), lambda i: (0, 0, 0)),
        out_shape=jax.ShapeDtypeStruct((2, _G, _H), jnp.float32),
    )(y, st, g8, be8, batch_t)


def _tc_head(pool, W1p, b1p, W2p, b2p):
    def body(po_ref, w1_ref, b1_ref, w2_ref, b2_ref, o_ref):
        pooled = po_ref[0] / jnp.maximum(po_ref[1], 1.0)
        hid = jnp.maximum(
            jnp.dot(pooled, w1_ref[...], preferred_element_type=jnp.float32) + b1_ref[...], 0.0)
        logit = jnp.dot(hid, w2_ref[...], preferred_element_type=jnp.float32) + b2_ref[...]
        o_ref[...] = jax.nn.sigmoid(logit)

    return pl.pallas_call(
        body,
        out_shape=jax.ShapeDtypeStruct((_G, _H), jnp.float32),
    )(pool, W1p, b1p, W2p, b2p)


def kernel(x, edge_index, batch, W_enc, b_enc, Wc0, bc0, Wc1, bc1,
           g0, be0, g1, be1, Wl1, bl1, Wl2, bl2):
    x_pad = jnp.pad(x, ((0, _NPAD - _N), (0, 0)))
    pad_e = _EPAD - _E
    src_t = jnp.concatenate(
        [edge_index[0], jnp.full((pad_e,), _N, jnp.int32)]).reshape(_NW, _CPT, _CHUNK)
    dst_t = jnp.concatenate(
        [edge_index[1], jnp.full((pad_e,), _N, jnp.int32)]).reshape(_NW, _CPT, _CHUNK)
    batch_t = jnp.concatenate(
        [batch, jnp.full((_NPAD - _N,), _G, jnp.int32)]).reshape(_NBLK, 1, _ROWS)
    ones_deg = jnp.ones((_CHUNK, _DEGW), jnp.float32)
    zeros_deg = jnp.zeros((_RPT, _DEGW), jnp.float32)
    b_enc8 = jnp.broadcast_to(b_enc, (8, _H))
    bc0_8 = jnp.broadcast_to(bc0, (8, _H))
    bc1_8 = jnp.broadcast_to(bc1, (8, _H))
    g0_8 = jnp.broadcast_to(g0, (8, _H))
    be0_8 = jnp.broadcast_to(be0, (8, _H))
    g1_8 = jnp.broadcast_to(g1, (8, _H))
    be1_8 = jnp.broadcast_to(be1, (8, _H))
    W1p = jnp.pad(Wl1, ((0, 0), (0, _H - _MID)))
    b1p = jnp.pad(jnp.broadcast_to(bl1, (_G, _MID)), ((0, 0), (0, _H - _MID)))
    W2p = jnp.pad(Wl2, ((0, _H - _MID), (0, _H - 1)))
    b2p = jnp.broadcast_to(bl2, (_G, _H))

    xh = _tc_encode_mm(x_pad, W_enc, b_enc8, Wc0)
    dp = _sc_degree(dst_t, ones_deg, zeros_deg)
    xs0 = _tc_scale_split(xh, dp)
    p0 = _sc_scatter(xs0, src_t, dst_t)
    y0, st0 = _tc_post(p0, dp, bc0_8)
    xs1 = _tc_norm_mm(y0, st0, g0_8, be0_8, Wc1, dp)
    p1 = _sc_scatter(xs1, src_t, dst_t)
    y1, st1 = _tc_post(p1, dp, bc1_8)
    pool = _tc_norm_pool(y1, st1, g1_8, be1_8, batch_t)
    out = _tc_head(pool, W1p, b1p, W2p, b2p)
    return out[:, 0]
